# Initial kernel scaffold; baseline (speedup 1.0000x reference)
import jax
import jax.numpy as jnp
from jax.experimental import pallas as pl


def kernel(x, lut):
    raise NotImplementedError("write your pallas kernel here")


# speedup vs baseline: 3.6713x; 3.6713x over previous
import functools
import math

import jax
import jax.numpy as jnp
from jax import lax
from jax.experimental import pallas as pl
from jax.experimental.pallas import tpu as pltpu
from jax.experimental.pallas import tpu_sc as plsc

_D_MODEL = 100000
_D = 64
_SCALE = math.sqrt(_D_MODEL)
_NC, _NS = 2, 16
_NW = _NC * _NS
_B = 4096 * 200
_B_PER_W = _B // _NW
_CHUNK = 512
_NCHUNK = _B_PER_W // _CHUNK


def _scale_body(lut_ref, out_ref):
    out_ref[...] = lut_ref[...] * _SCALE


_scale_call = pl.pallas_call(
    _scale_body,
    grid=(25,),
    in_specs=[pl.BlockSpec((_D_MODEL // 25, _D), lambda i: (i, 0))],
    out_specs=pl.BlockSpec((_D_MODEL // 25, _D), lambda i: (i, 0)),
    out_shape=jax.ShapeDtypeStruct((_D_MODEL, _D), jnp.float32),
)

_mesh = plsc.VectorSubcoreMesh(core_axis_name="c", subcore_axis_name="s")


@functools.partial(
    pl.kernel,
    out_type=jax.ShapeDtypeStruct((_B, _D), jnp.float32),
    mesh=_mesh,
    scratch_types=[
        pltpu.VMEM((_CHUNK,), jnp.int32),
        pltpu.VMEM((_CHUNK, _D), jnp.float32),
        pltpu.SemaphoreType.DMA,
    ],
    compiler_params=pltpu.CompilerParams(use_tc_tiling_on_sc=False),
)
def _gather_kernel(table_hbm, idx_hbm, out_hbm, idx_v, rows_v, sem):
    wid = lax.axis_index("s") * _NC + lax.axis_index("c")
    base = wid * _B_PER_W

    def chunk_body(g, carry):
        off = base + g * _CHUNK
        pltpu.sync_copy(idx_hbm.at[pl.ds(off, _CHUNK)], idx_v)
        pltpu.async_copy(table_hbm.at[idx_v], rows_v, sem).wait()
        pltpu.sync_copy(rows_v, out_hbm.at[pl.ds(off, _CHUNK)])
        return carry

    lax.fori_loop(0, _NCHUNK, chunk_body, 0)


def kernel(x, lut):
    scaled = _scale_call(lut)
    idx = x.reshape(-1).astype(jnp.int32)
    out = _gather_kernel(scaled, idx)
    return out.reshape(x.shape + (_D,))


# speedup vs baseline: 3.9228x; 1.0685x over previous
import functools
import math

import jax
import jax.numpy as jnp
from jax import lax
from jax.experimental import pallas as pl
from jax.experimental.pallas import tpu as pltpu
from jax.experimental.pallas import tpu_sc as plsc

_D_MODEL = 100000
_D = 64
_SCALE = math.sqrt(_D_MODEL)
_NC, _NS = 2, 16
_NW = _NC * _NS
_B = 4096 * 200
_B_PER_W = _B // _NW
_CHUNK = 800
_NCHUNK = _B_PER_W // _CHUNK


def _scale_body(lut_ref, out_ref):
    out_ref[...] = lut_ref[...] * _SCALE


_scale_call = pl.pallas_call(
    _scale_body,
    grid=(25,),
    in_specs=[pl.BlockSpec((_D_MODEL // 25, _D), lambda i: (i, 0))],
    out_specs=pl.BlockSpec((_D_MODEL // 25, _D), lambda i: (i, 0)),
    out_shape=jax.ShapeDtypeStruct((_D_MODEL, _D), jnp.float32),
)

_mesh = plsc.VectorSubcoreMesh(core_axis_name="c", subcore_axis_name="s")


@functools.partial(
    pl.kernel,
    out_type=jax.ShapeDtypeStruct((_B, _D), jnp.float32),
    mesh=_mesh,
    scratch_types=[
        pltpu.VMEM((_B_PER_W,), jnp.int32),
        pltpu.VMEM((_CHUNK, _D), jnp.float32),
        pltpu.VMEM((_CHUNK, _D), jnp.float32),
        pltpu.SemaphoreType.DMA,
        pltpu.SemaphoreType.DMA,
        pltpu.SemaphoreType.DMA,
        pltpu.SemaphoreType.DMA,
    ],
    compiler_params=pltpu.CompilerParams(use_tc_tiling_on_sc=False),
)
def _gather_kernel(table_hbm, idx_hbm, out_hbm, idx_v, rows0, rows1,
                   gsem0, gsem1, osem0, osem1):
    wid = lax.axis_index("s") * _NC + lax.axis_index("c")
    base = wid * _B_PER_W
    rows = (rows0, rows1)
    gsem = (gsem0, gsem1)
    osem = (osem0, osem1)

    pltpu.sync_copy(idx_hbm.at[pl.ds(base, _B_PER_W)], idx_v)

    def start_gather(g, b):
        pltpu.async_copy(
            table_hbm.at[idx_v.at[pl.ds(g * _CHUNK, _CHUNK)]], rows[b], gsem[b])

    def wait_gather(g, b):
        pltpu.make_async_copy(
            table_hbm.at[idx_v.at[pl.ds(g * _CHUNK, _CHUNK)]], rows[b], gsem[b]
        ).wait()

    def start_out(g, b):
        pltpu.async_copy(
            rows[b], out_hbm.at[pl.ds(base + g * _CHUNK, _CHUNK)], osem[b])

    def wait_out(g, b):
        pltpu.make_async_copy(
            rows[b], out_hbm.at[pl.ds(base + g * _CHUNK, _CHUNK)], osem[b]
        ).wait()

    start_gather(0, 0)
    wait_gather(0, 0)
    start_gather(1, 1)
    start_out(0, 0)

    wait_gather(1, 1)
    wait_out(0, 0)
    start_gather(2, 0)
    start_out(1, 1)

    def pair(k, carry):
        g = 2 * k
        wait_gather(g, 0)
        wait_out(g - 1, 1)
        start_gather(g + 1, 1)
        start_out(g, 0)
        wait_gather(g + 1, 1)
        wait_out(g, 0)
        start_gather(g + 2, 0)
        start_out(g + 1, 1)
        return carry

    lax.fori_loop(1, _NCHUNK // 2 - 1, pair, 0)

    g = _NCHUNK - 2
    wait_gather(g, 0)
    wait_out(g - 1, 1)
    start_gather(g + 1, 1)
    start_out(g, 0)

    wait_gather(g + 1, 1)
    start_out(g + 1, 1)
    wait_out(g, 0)
    wait_out(g + 1, 1)


def kernel(x, lut):
    scaled = _scale_call(lut)
    idx = x.reshape(-1).astype(jnp.int32)
    out = _gather_kernel(scaled, idx)
    return out.reshape(x.shape + (_D,))
